# Initial kernel scaffold; baseline (speedup 1.0000x reference)
import jax
import jax.numpy as jnp
from jax.experimental import pallas as pl


def kernel(elem_hiddens, Wq_idx, bq_idx, key0, key1, key2, key3, mem_keys, mem_vals, Wq_mha, bq_mha, W1, b1, W2, b2, ln_g, ln_b):
    raise NotImplementedError("write your pallas kernel here")


# speedup vs baseline: 3.9022x; 3.9022x over previous
import jax
import jax.numpy as jnp
from jax import lax
from jax.experimental import pallas as pl

TOKB = 256
K = 10
NC = 100
F32 = jnp.float32


def _take(a, idx):
    return jnp.take_along_axis(a, idx, axis=-1, mode="promise_in_bounds")


def _tree10(t):
    w = [t[0] + t[8] + t[4], t[1] + t[9] + t[5], t[2] + t[6], t[3] + t[7]]
    return (w[0] + w[2]) + (w[1] + w[3])


def _tree10_lanes(v):
    return _tree10([v[:, i:i + 1] for i in range(10)])


def _tree128_lanes(v):
    cur = jnp.concatenate([v, jnp.zeros((v.shape[0], 28), F32)], -1)
    w = 128
    while w > 1:
        w //= 2
        cur = cur[:, :w] + cur[:, w:2 * w]
    return cur


def _topk10(work):
    iota = lax.broadcasted_iota(jnp.int32, work.shape, 1)
    big = jnp.int32(work.shape[1])
    vs, js = [], []
    for _ in range(K):
        m = jnp.max(work, axis=-1, keepdims=True)
        idx = jnp.min(jnp.where(work == m, iota, big), axis=-1, keepdims=True)
        work = jnp.where(iota == idx, -jnp.inf, work)
        vs.append(m)
        js.append(idx)
    return jnp.concatenate(vs, -1), jnp.concatenate(js, -1)


def _body(x_ref, wqi_ref, bqi_ref, wqm_ref, bqm_ref, k0t_ref, kc1t_ref,
          kc2t_ref, kc3t_ref, mv_ref, w1_ref, b1_ref, w2_ref,
          b2_ref, lng_ref, lnb_ref, o_ref):
    x = x_ref[...]
    T = x.shape[0]
    qidx = jnp.dot(x, wqi_ref[...], preferred_element_type=F32) + bqi_ref[0:1, :]
    qmha = jnp.dot(x, wqm_ref[...], preferred_element_type=F32) + bqm_ref[0:1, :]

    s0 = jnp.dot(qidx[:, 0:64], k0t_ref[...], preferred_element_type=F32,
                 precision=lax.Precision.HIGHEST) * 0.125
    e0 = jnp.exp(s0 - jnp.max(s0, -1, keepdims=True))
    p0 = e0 / _tree128_lanes(e0)
    valraw, ind = _topk10(p0)

    for lvl, kct_ref in ((1, kc1t_ref), (2, kc2t_ref), (3, kc3t_ref)):
        q = qidx[:, 64 * lvl:64 * lvl + 64]
        s = jnp.dot(q, kct_ref[...], preferred_element_type=F32,
                    precision=lax.Precision.HIGHEST) * 0.125
        parts = [_take(s[:, 128 * r:128 * r + 128], ind) for r in range(10)]
        gmax = parts[0]
        for pr in parts[1:]:
            gmax = jnp.maximum(gmax, pr)
        eparts = [jnp.exp(pr - gmax) for pr in parts]
        den = _tree10(eparts)
        valn = valraw / _tree10_lanes(valraw)
        attn = jnp.concatenate([valn * (ep / den) for ep in eparts], -1)
        valraw, cprime = _topk10(attn)
        ind = (cprime % 10) * 10 + cprime // 10

    oh = (ind[:, :, None]
          == lax.broadcasted_iota(jnp.int32, (T, K, NC), 2)).astype(F32)
    gv = jnp.dot(oh.reshape(T * K, NC), mv_ref[...], preferred_element_type=F32,
                 precision=lax.Precision.HIGHEST).reshape(T, K, 256)

    parts = []
    for r in range(4):
        gvr = gv[:, :, 64 * r:64 * r + 64]
        qmr = qmha[:, 64 * r:64 * r + 64]
        sr = jnp.sum(gvr * qmr[:, None, :], -1) * 0.125
        er = jnp.exp(sr - jnp.max(sr, -1, keepdims=True))
        pr = er / jnp.sum(er, -1, keepdims=True)
        parts.append(jnp.sum(gvr * pr[:, :, None], axis=1))
    attd = jnp.concatenate(parts, -1)

    h = jnp.dot(attd, w1_ref[...], preferred_element_type=F32) + b1_ref[0:1, :]
    h = 0.5 * h * (1.0 + lax.erf(h * 0.7071067811865476))
    h = jnp.dot(h, w2_ref[...], preferred_element_type=F32) + b2_ref[0:1, :]
    mu = jnp.mean(h, -1, keepdims=True)
    hc = h - mu
    var = jnp.mean(hc * hc, -1, keepdims=True)
    o_ref[...] = lng_ref[0:1, :] * hc / jnp.sqrt(var + 1e-5) + lnb_ref[0:1, :]


def kernel(elem_hiddens, Wq_idx, bq_idx, key0, key1, key2, key3, mem_keys,
           mem_vals, Wq_mha, bq_mha, W1, b1, W2, b2, ln_g, ln_b):
    B, N, HID = elem_hiddens.shape
    T = B * N
    xf = elem_hiddens.reshape(T, HID)
    k0t = key0.T

    def rmaj(kl):
        krm = jnp.transpose(kl, (1, 0, 2))
        krm = jnp.pad(krm, ((0, 0), (0, 28), (0, 0)))
        return krm.reshape(1280, 64).T

    kc1t = rmaj(key1)
    kc2t = rmaj(key2[:NC])
    kc3t = rmaj(key3[:NC])
    mv = jnp.transpose(mem_vals[:, :NC, :], (1, 0, 2)).reshape(NC, 256)
    t8 = lambda v: jnp.tile(v[None, :], (8, 1))

    grid = T // TOKB
    cst = lambda shape: pl.BlockSpec(shape, lambda i: (0, 0))
    out = pl.pallas_call(
        _body,
        grid=(grid,),
        in_specs=[
            pl.BlockSpec((TOKB, HID), lambda i: (i, 0)),
            cst((HID, 256)), cst((8, 256)),
            cst((HID, 256)), cst((8, 256)),
            cst((64, 100)), cst((64, 1280)), cst((64, 1280)), cst((64, 1280)),
            cst((NC, 256)),
            cst((256, 3072)), cst((8, 3072)),
            cst((3072, HID)), cst((8, HID)),
            cst((8, HID)), cst((8, HID)),
        ],
        out_specs=pl.BlockSpec((TOKB, HID), lambda i: (i, 0)),
        out_shape=jax.ShapeDtypeStruct((T, HID), F32),
    )(xf, Wq_idx, t8(bq_idx), Wq_mha, t8(bq_mha), k0t, kc1t, kc2t, kc3t,
      mv, W1, t8(b1), W2, t8(b2), t8(ln_g), t8(ln_b))
    return out.reshape(B, N, HID)
